# baseline (speedup 1.0000x reference)
import functools

import jax
import jax.numpy as jnp
from jax import lax
from jax.experimental import pallas as pl
from jax.experimental.pallas import tpu as pltpu
from jax.experimental.pallas import tpu_sc as plsc

LANES = 16


def _pos_embed_body(nb, n_chan, h, w, d, cpw, nc,
                    g_hbm, g16_hbm, out_hbm, g_vmem, g16_vmem,
                    vol0, vol1, sem0, sem1):
    q = w * d
    vol_len = h * q
    vregs_per_wblock = d // LANES

    wid = lax.axis_index("s") * nc + lax.axis_index("c")
    base = wid * cpw
    vols = (vol0, vol1)
    sems = (sem0, sem1)

    for i in range(cpw):
        p = i % 2
        vol = vols[p]
        sem = sems[p]
        c = base + i

        if i >= 2:
            for _ in range(nb):
                pltpu.make_async_copy(vol, out_hbm.at[pl.ds(0, vol_len)], sem).wait()

        pltpu.sync_copy(g_hbm.at[c], g_vmem)
        pltpu.sync_copy(g16_hbm.at[c], g16_vmem)

        seg = c // (n_chan // 3)

        @pl.when(seg == 0)
        def _():
            def per_w(wi, carry):
                v = g16_vmem[pl.ds(wi * LANES, LANES)]
                for hi in range(h):
                    for j in range(vregs_per_wblock):
                        vol[pl.ds(hi * q + wi * d + j * LANES, LANES)] = v
                return carry
            lax.fori_loop(0, w, per_w, 0)

        @pl.when(seg == 1)
        def _():
            def per_h(hi, carry):
                v = g16_vmem[pl.ds(hi * LANES, LANES)]
                for j in range(q // LANES):
                    vol[pl.ds(hi * q + j * LANES, LANES)] = v
                return carry
            lax.fori_loop(0, h, per_h, 0)

        @pl.when(seg == 2)
        def _():
            gv = [g_vmem[pl.ds(j * LANES, LANES)] for j in range(d // LANES)]
            def per_h(hi, carry):
                for j in range(q // LANES):
                    vol[pl.ds(hi * q + j * LANES, LANES)] = gv[j % len(gv)]
                return carry
            lax.fori_loop(0, h, per_h, 0)

        for b in range(nb):
            pltpu.async_copy(
                vol, out_hbm.at[pl.ds((b * n_chan + c) * vol_len, vol_len)], sem)

    for i in range(min(2, cpw)):
        for _ in range(nb):
            pltpu.make_async_copy(
                vols[i], out_hbm.at[pl.ds(0, vol_len)], sems[i]).wait()


def kernel(tensor_list, row_embed_weight, col_embed_weight, depth_embed_weight):
    x = tensor_list
    h, w, d = x.shape[-3], x.shape[-2], x.shape[-1]
    nb = x.shape[0]
    f = row_embed_weight.shape[-1]
    n_chan = 3 * f
    vol_len = h * w * d

    g = jnp.concatenate(
        [col_embed_weight[:w].T, row_embed_weight[:h].T, depth_embed_weight[:d].T],
        axis=0,
    )
    g16 = jnp.repeat(g, LANES, axis=1)

    info = plsc.get_sparse_core_info()
    nc, ns = info.num_cores, info.num_subcores
    nw = nc * ns
    cpw = n_chan // nw

    run = pl.kernel(
        functools.partial(_pos_embed_body, nb, n_chan, h, w, d, cpw, nc),
        mesh=plsc.VectorSubcoreMesh(core_axis_name="c", subcore_axis_name="s"),
        out_type=jax.ShapeDtypeStruct((nb * n_chan * vol_len,), jnp.float32),
        scratch_types=[
            pltpu.VMEM((w,), jnp.float32),
            pltpu.VMEM((w * LANES,), jnp.float32),
            pltpu.VMEM((vol_len,), jnp.float32),
            pltpu.VMEM((vol_len,), jnp.float32),
            pltpu.SemaphoreType.DMA,
            pltpu.SemaphoreType.DMA,
        ],
    )
    out = run(g, g16)
    return out.reshape(nb, n_chan, h, w, d)


# speedup vs baseline: 6.9548x; 6.9548x over previous
import functools

import jax
import jax.numpy as jnp
from jax import lax
from jax.experimental import pallas as pl
from jax.experimental.pallas import tpu as pltpu
from jax.experimental.pallas import tpu_sc as plsc

LANES = 16
SUB = 8
LN = 128
TILE = SUB * LN


def _pos_embed_body(nb, h, w, d, f, nc,
                    colf_hbm, rowf_hbm, depf_hbm, out_hbm,
                    colv, rowv, depv, slab0, slab1, sem0, sem1):
    dt_n = d // SUB
    ct_n = (3 * f) // LN
    slab_len = dt_n * ct_n * TILE

    hh = lax.axis_index("s") * nc + lax.axis_index("c")

    pltpu.sync_copy(colf_hbm, colv)
    pltpu.sync_copy(depf_hbm, depv)
    pltpu.sync_copy(rowf_hbm.at[pl.ds(hh * f, f)], rowv)

    row_regs = [rowv[pl.ds(j * LANES, LANES)] for j in range(f // LANES)]

    def build_slab(buf, wq):
        col_regs = [colv[pl.ds(wq * f + j * LANES, LANES)]
                    for j in range(f // LANES)]
        for dt in range(dt_n):
            b0 = (dt * ct_n) * TILE
            b1 = (dt * ct_n + 1) * TILE
            b2 = (dt * ct_n + 2) * TILE
            for dr in range(SUB):
                for j in range(LN // LANES):
                    o = dr * LN + j * LANES
                    buf[pl.ds(b0 + o, LANES)] = col_regs[j]
                    buf[pl.ds(b1 + o, LANES)] = row_regs[j]
                    buf[pl.ds(b2 + o, LANES)] = depv[pl.ds(dt * TILE + o, LANES)]

    def fire(buf, wq, sem):
        for b in range(nb):
            off = ((b * h + hh) * w + wq) * slab_len
            pltpu.async_copy(buf, out_hbm.at[pl.ds(off, slab_len)], sem)

    def drain(buf, sem):
        for _ in range(nb):
            pltpu.make_async_copy(
                buf, out_hbm.at[pl.ds(0, slab_len)], sem).wait()

    build_slab(slab0, 0)
    fire(slab0, 0, sem0)
    build_slab(slab1, 1)
    fire(slab1, 1, sem1)

    def pair(k, carry):
        wq = 2 * k
        drain(slab0, sem0)
        build_slab(slab0, wq)
        fire(slab0, wq, sem0)
        drain(slab1, sem1)
        build_slab(slab1, wq + 1)
        fire(slab1, wq + 1, sem1)
        return carry

    lax.fori_loop(1, w // 2, pair, 0)

    drain(slab0, sem0)
    drain(slab1, sem1)


def kernel(tensor_list, row_embed_weight, col_embed_weight, depth_embed_weight):
    x = tensor_list
    h, w, d = x.shape[-3], x.shape[-2], x.shape[-1]
    nb = x.shape[0]
    f = row_embed_weight.shape[-1]
    n_chan = 3 * f

    info = plsc.get_sparse_core_info()
    nc, ns = info.num_cores, info.num_subcores
    assert nc * ns == h, "one vector subcore per h plane"

    colf = col_embed_weight[:w].reshape(-1)
    rowf = row_embed_weight[:h].reshape(-1)
    depf = depth_embed_weight[:d].reshape(-1)

    dt_n = d // SUB
    ct_n = n_chan // LN
    total = nb * h * w * dt_n * ct_n * TILE

    run = pl.kernel(
        functools.partial(_pos_embed_body, nb, h, w, d, f, nc),
        mesh=plsc.VectorSubcoreMesh(core_axis_name="c", subcore_axis_name="s"),
        out_type=jax.ShapeDtypeStruct((total,), jnp.float32),
        scratch_types=[
            pltpu.VMEM((w * f,), jnp.float32),
            pltpu.VMEM((f,), jnp.float32),
            pltpu.VMEM((d * f,), jnp.float32),
            pltpu.VMEM((dt_n * ct_n * TILE,), jnp.float32),
            pltpu.VMEM((dt_n * ct_n * TILE,), jnp.float32),
            pltpu.SemaphoreType.DMA,
            pltpu.SemaphoreType.DMA,
        ],
    )
    out = run(colf, rowf, depf)
    out7 = out.reshape(nb, h, w, dt_n, ct_n, SUB, LN)
    out5 = out7.transpose(0, 4, 6, 1, 2, 3, 5).reshape(nb, n_chan, h, w, d)
    return out5


# speedup vs baseline: 8.0294x; 1.1545x over previous
import functools

import jax
import jax.numpy as jnp
from jax import lax
from jax.experimental import pallas as pl
from jax.experimental.pallas import tpu as pltpu
from jax.experimental.pallas import tpu_sc as plsc

LANES = 16
SUB = 8
LN = 128
TILE = SUB * LN


def _pos_embed_body(nb, h, w, d, f, nc,
                    colf_hbm, rowf_hbm, depf_hbm, out_hbm,
                    colv, rowv, depv, rowt, colt0, colt1,
                    semc0, semc1, semrd):
    dt_n = d // SUB
    ct_n = (3 * f) // LN
    slab_len = dt_n * ct_n * TILE
    jn = f // LANES

    hh = lax.axis_index("s") * nc + lax.axis_index("c")

    pltpu.sync_copy(colf_hbm.at[pl.ds(0, w * f)], colv)
    pltpu.sync_copy(depf_hbm.at[pl.ds(0, d * f)], depv)
    pltpu.sync_copy(rowf_hbm.at[pl.ds(hh * f, f)], rowv)

    row_regs = [rowv[pl.ds(j * LANES, LANES)] for j in range(jn)]
    for dr in range(SUB):
        for j in range(jn):
            rowt[pl.ds(dr * LN + j * LANES, LANES)] = row_regs[j]

    def build_col(buf, wq):
        col_regs = [colv[pl.ds(wq * f + j * LANES, LANES)] for j in range(jn)]
        for dr in range(SUB):
            for j in range(jn):
                buf[pl.ds(dr * LN + j * LANES, LANES)] = col_regs[j]

    def fire(colt, wq, semc):
        for b in range(nb):
            base = ((b * h + hh) * w + wq) * slab_len
            for dt in range(dt_n):
                off = base + dt * ct_n * TILE
                pltpu.async_copy(colt, out_hbm.at[pl.ds(off, TILE)], semc)
                pltpu.async_copy(rowt, out_hbm.at[pl.ds(off + TILE, TILE)],
                                 semrd)
                pltpu.async_copy(depv.at[pl.ds(dt * TILE, TILE)],
                                 out_hbm.at[pl.ds(off + 2 * TILE, TILE)],
                                 semrd)

    def drain(buf, sem, n):
        for _ in range(n):
            pltpu.make_async_copy(buf, out_hbm.at[pl.ds(0, TILE)], sem).wait()

    col_fires = nb * dt_n

    build_col(colt0, 0)
    fire(colt0, 0, semc0)
    build_col(colt1, 1)
    fire(colt1, 1, semc1)

    def pair(k, carry):
        wq = 2 * k
        drain(colt0, semc0, col_fires)
        build_col(colt0, wq)
        fire(colt0, wq, semc0)
        drain(colt1, semc1, col_fires)
        build_col(colt1, wq + 1)
        fire(colt1, wq + 1, semc1)
        return carry

    lax.fori_loop(1, w // 2, pair, 0)

    drain(colt0, semc0, col_fires)
    drain(colt1, semc1, col_fires)

    def dw(i, carry):
        pltpu.make_async_copy(rowt, out_hbm.at[pl.ds(0, TILE)], semrd).wait()
        return carry
    lax.fori_loop(0, 2 * w * nb * dt_n, dw, 0)


def kernel(tensor_list, row_embed_weight, col_embed_weight, depth_embed_weight):
    x = tensor_list
    h, w, d = x.shape[-3], x.shape[-2], x.shape[-1]
    nb = x.shape[0]
    f = row_embed_weight.shape[-1]
    n_chan = 3 * f

    info = plsc.get_sparse_core_info()
    nc, ns = info.num_cores, info.num_subcores
    assert nc * ns == h, "one vector subcore per h plane"

    colf = col_embed_weight.reshape(-1)
    rowf = row_embed_weight.reshape(-1)
    depf = depth_embed_weight.reshape(-1)

    dt_n = d // SUB
    ct_n = n_chan // LN
    total = nb * h * w * dt_n * ct_n * TILE

    run = pl.kernel(
        functools.partial(_pos_embed_body, nb, h, w, d, f, nc),
        mesh=plsc.VectorSubcoreMesh(core_axis_name="c", subcore_axis_name="s"),
        out_type=jax.ShapeDtypeStruct((total,), jnp.float32),
        scratch_types=[
            pltpu.VMEM((w * f,), jnp.float32),
            pltpu.VMEM((f,), jnp.float32),
            pltpu.VMEM((d * f,), jnp.float32),
            pltpu.VMEM((TILE,), jnp.float32),
            pltpu.VMEM((TILE,), jnp.float32),
            pltpu.VMEM((TILE,), jnp.float32),
            pltpu.SemaphoreType.DMA,
            pltpu.SemaphoreType.DMA,
            pltpu.SemaphoreType.DMA,
        ],
    )
    out = run(colf, rowf, depf)
    out7 = out.reshape(nb, h, w, dt_n, ct_n, SUB, LN)
    out5 = out7.transpose(0, 4, 6, 1, 2, 3, 5).reshape(nb, n_chan, h, w, d)
    return out5
